# initial kernel scaffold (unmeasured)
import jax
import jax.numpy as jnp
from jax import lax
from jax.experimental import pallas as pl
from jax.experimental.pallas import tpu as pltpu

N_Y = 4


def kernel(Q, K, V):
    b, s_loc, nh, d = Q.shape
    scale = d ** -0.5

    def body(q_ref, k_ref, v_ref, o_ref, kbuf, vbuf, send_sems, recv_sems):
        my_x = lax.axis_index("x")
        my_y = lax.axis_index("y")
        my_z = lax.axis_index("z")
        right = (my_y + 1) % N_Y
        left = (my_y - 1) % N_Y

        barrier_sem = pltpu.get_barrier_semaphore()
        for nbr in (left, right):
            pl.semaphore_signal(
                barrier_sem,
                inc=1,
                device_id=(my_x, nbr, my_z),
                device_id_type=pl.DeviceIdType.MESH,
            )
        pl.semaphore_wait(barrier_sem, 2)

        kbuf[0] = k_ref[...].astype(jnp.bfloat16)
        vbuf[0] = v_ref[...].astype(jnp.bfloat16)

        for h in range(N_Y - 1):
            rk = pltpu.make_async_remote_copy(
                src_ref=kbuf.at[h],
                dst_ref=kbuf.at[h + 1],
                send_sem=send_sems.at[h, 0],
                recv_sem=recv_sems.at[h, 0],
                device_id=(my_x, right, my_z),
                device_id_type=pl.DeviceIdType.MESH,
            )
            rv = pltpu.make_async_remote_copy(
                src_ref=vbuf.at[h],
                dst_ref=vbuf.at[h + 1],
                send_sem=send_sems.at[h, 1],
                recv_sem=recv_sems.at[h, 1],
                device_id=(my_x, right, my_z),
                device_id_type=pl.DeviceIdType.MESH,
            )
            rk.start()
            rv.start()
            rk.wait()
            rv.wait()

        for bb in range(b):
            for hh in range(nh):
                q = q_ref[bb, :, hh, :].astype(jnp.bfloat16)
                s_parts = [
                    lax.dot_general(
                        q,
                        kbuf[o, bb, :, hh, :],
                        (((1,), (1,)), ((), ())),
                        preferred_element_type=jnp.float32,
                    )
                    for o in range(N_Y)
                ]
                s = jnp.concatenate(s_parts, axis=1) * scale
                m = jnp.max(s, axis=1, keepdims=True)
                p = jnp.exp(s - m)
                p = p / jnp.sum(p, axis=1, keepdims=True)
                pb = p.astype(jnp.bfloat16)
                acc = lax.dot_general(
                    pb[:, 0:s_loc],
                    vbuf[0, bb, :, hh, :],
                    (((1,), (0,)), ((), ())),
                    preferred_element_type=jnp.float32,
                )
                for o in range(1, N_Y):
                    acc += lax.dot_general(
                        pb[:, o * s_loc:(o + 1) * s_loc],
                        vbuf[o, bb, :, hh, :],
                        (((1,), (0,)), ((), ())),
                        preferred_element_type=jnp.float32,
                    )
                o_ref[bb, :, hh, :] = acc

    return pl.pallas_call(
        body,
        out_shape=jax.ShapeDtypeStruct((b, s_loc, nh, d), jnp.float32),
        in_specs=[pl.BlockSpec(memory_space=pltpu.VMEM)] * 3,
        out_specs=pl.BlockSpec(memory_space=pltpu.VMEM),
        scratch_shapes=[
            pltpu.VMEM((N_Y, b, s_loc, nh, d), jnp.bfloat16),
            pltpu.VMEM((N_Y, b, s_loc, nh, d), jnp.bfloat16),
            pltpu.SemaphoreType.DMA((N_Y - 1, 2)),
            pltpu.SemaphoreType.DMA((N_Y - 1, 2)),
        ],
        compiler_params=pltpu.CompilerParams(collective_id=0),
    )(Q, K, V)


# baseline (device time: 215022 ns/iter reference)
import jax
import jax.numpy as jnp
from jax import lax
from jax.experimental import pallas as pl
from jax.experimental.pallas import tpu as pltpu

N_Y = 4


def kernel(Q, K, V):
    b, s_loc, nh, d = Q.shape
    hd = nh * d
    scale = d ** -0.5

    def body(q_ref, k_ref, v_ref, o_ref, kbuf, vbuf, send_sems, recv_sems):
        my_x = lax.axis_index("x")
        my_y = lax.axis_index("y")
        my_z = lax.axis_index("z")
        right = (my_y + 1) % N_Y
        left = (my_y - 1) % N_Y

        barrier_sem = pltpu.get_barrier_semaphore()
        for nbr in (left, right):
            pl.semaphore_signal(
                barrier_sem,
                inc=1,
                device_id=(my_x, nbr, my_z),
                device_id_type=pl.DeviceIdType.MESH,
            )
        pl.semaphore_wait(barrier_sem, 2)

        kbuf[0] = k_ref[...].astype(jnp.bfloat16)
        vbuf[0] = v_ref[...].astype(jnp.bfloat16)

        for h in range(N_Y - 1):
            rk = pltpu.make_async_remote_copy(
                src_ref=kbuf.at[h],
                dst_ref=kbuf.at[h + 1],
                send_sem=send_sems.at[h, 0],
                recv_sem=recv_sems.at[h, 0],
                device_id=(my_x, right, my_z),
                device_id_type=pl.DeviceIdType.MESH,
            )
            rv = pltpu.make_async_remote_copy(
                src_ref=vbuf.at[h],
                dst_ref=vbuf.at[h + 1],
                send_sem=send_sems.at[h, 1],
                recv_sem=recv_sems.at[h, 1],
                device_id=(my_x, right, my_z),
                device_id_type=pl.DeviceIdType.MESH,
            )
            rk.start()
            rv.start()
            rk.wait()
            rv.wait()

        for bb in range(b):
            for hh in range(nh):
                lo, hi = hh * d, (hh + 1) * d
                q = q_ref[bb, :, lo:hi].astype(jnp.bfloat16)
                s_parts = [
                    lax.dot_general(
                        q,
                        kbuf[o, bb, :, lo:hi],
                        (((1,), (1,)), ((), ())),
                        preferred_element_type=jnp.float32,
                    )
                    for o in range(N_Y)
                ]
                s = jnp.concatenate(s_parts, axis=1) * scale
                m = jnp.max(s, axis=1, keepdims=True)
                p = jnp.exp(s - m)
                p = p / jnp.sum(p, axis=1, keepdims=True)
                pb = p.astype(jnp.bfloat16)
                acc = lax.dot_general(
                    pb[:, 0:s_loc],
                    vbuf[0, bb, :, lo:hi],
                    (((1,), (0,)), ((), ())),
                    preferred_element_type=jnp.float32,
                )
                for o in range(1, N_Y):
                    acc += lax.dot_general(
                        pb[:, o * s_loc:(o + 1) * s_loc],
                        vbuf[o, bb, :, lo:hi],
                        (((1,), (0,)), ((), ())),
                        preferred_element_type=jnp.float32,
                    )
                o_ref[bb, :, lo:hi] = acc

    out = pl.pallas_call(
        body,
        out_shape=jax.ShapeDtypeStruct((b, s_loc, hd), jnp.float32),
        in_specs=[pl.BlockSpec(memory_space=pltpu.VMEM)] * 3,
        out_specs=pl.BlockSpec(memory_space=pltpu.VMEM),
        scratch_shapes=[
            pltpu.VMEM((N_Y, b, s_loc, hd), jnp.bfloat16),
            pltpu.VMEM((N_Y, b, s_loc, hd), jnp.bfloat16),
            pltpu.SemaphoreType.DMA((N_Y - 1, 2)),
            pltpu.SemaphoreType.DMA((N_Y - 1, 2)),
        ],
        compiler_params=pltpu.CompilerParams(
            collective_id=0,
            vmem_limit_bytes=100 * 1024 * 1024,
        ),
    )(
        Q.reshape(b, s_loc, hd),
        K.reshape(b, s_loc, hd),
        V.reshape(b, s_loc, hd),
    )
    return out.reshape(b, s_loc, nh, d)
